# Initial kernel scaffold; baseline (speedup 1.0000x reference)
import jax
import jax.numpy as jnp
from jax.experimental import pallas as pl


def kernel(input_ids, embedding, hard_norm_weight, proj_weight):
    raise NotImplementedError("write your pallas kernel here")


# speedup vs baseline: 2.9887x; 2.9887x over previous
import functools

import jax
import jax.numpy as jnp
from jax import lax
from jax.experimental import pallas as pl
from jax.experimental.pallas import tpu as pltpu
from jax.experimental.pallas import tpu_sc as plsc

_EPS = 1e-06
_D_OUT = 2048


def _table_body(emb_ref, nw_ref, pw_ref, out_ref):
    emb = emb_ref[...]
    normed = emb * lax.rsqrt(jnp.mean(emb * emb, axis=-1, keepdims=True) + _EPS)
    normed = normed * nw_ref[...]
    proj = lax.dot_general(
        normed, pw_ref[...], (((1,), (1,)), ((), ())),
        preferred_element_type=jnp.float32)
    out_ref[...] = proj * lax.rsqrt(
        jnp.mean(proj * proj, axis=-1, keepdims=True) + _EPS)


def _make_gather(vocab, d, batch):
    info = plsc.get_sparse_core_info()
    nc, ns = info.num_cores, info.num_subcores
    nw = nc * ns
    assert batch % (8 * nw) == 0
    b_per_w = batch // nw
    chunk = 16
    assert b_per_w % (2 * chunk) == 0
    n_pairs = b_per_w // (2 * chunk)
    mesh = plsc.VectorSubcoreMesh(core_axis_name="c", subcore_axis_name="s")

    @functools.partial(
        pl.kernel,
        mesh=mesh,
        out_type=jax.ShapeDtypeStruct((batch, d), jnp.float32),
        scratch_types=[
            pltpu.VMEM((b_per_w,), jnp.int32),
            pltpu.VMEM((chunk, d), jnp.float32),
            pltpu.VMEM((chunk, d), jnp.float32),
            pltpu.SemaphoreType.DMA,
            pltpu.SemaphoreType.DMA,
            pltpu.SemaphoreType.DMA,
            pltpu.SemaphoreType.DMA,
        ],
    )
    def gather(table_hbm, ids_hbm, out_hbm, idx_v, buf0, buf1,
               gsem0, gsem1, wsem0, wsem1):
        wid = lax.axis_index("s") * nc + lax.axis_index("c")
        base = wid * b_per_w
        pltpu.sync_copy(ids_hbm.at[pl.ds(base, b_per_w)], idx_v)

        def start_gather(i, buf, gsem):
            pltpu.async_copy(
                table_hbm.at[idx_v.at[pl.ds(i * chunk, chunk)]], buf, gsem)

        def wait_gather(buf, gsem):
            pltpu.make_async_copy(
                table_hbm.at[idx_v.at[pl.ds(0, chunk)]], buf, gsem).wait()

        start_gather(0, buf0, gsem0)
        start_gather(1, buf1, gsem1)

        def body(p, _):
            for par, (buf, gsem, wsem) in enumerate(
                    ((buf0, gsem0, wsem0), (buf1, gsem1, wsem1))):
                i = p * 2 + par
                wait_gather(buf, gsem)
                pltpu.async_copy(
                    buf, out_hbm.at[pl.ds(base + i * chunk, chunk)], wsem
                ).wait()

                @pl.when(i + 2 < 2 * n_pairs)
                def _():
                    start_gather(i + 2, buf, gsem)
            return 0

        lax.fori_loop(0, n_pairs, body, 0)

    return gather


def kernel(input_ids, embedding, hard_norm_weight, proj_weight):
    vocab, mm_hidden = embedding.shape
    b, s = input_ids.shape
    table = pl.pallas_call(
        _table_body,
        out_shape=jax.ShapeDtypeStruct((vocab, _D_OUT), jnp.float32),
    )(embedding, hard_norm_weight.reshape(1, mm_hidden), proj_weight)
    ids_flat = input_ids.reshape(b * s).astype(jnp.int32)
    gather = _make_gather(vocab, _D_OUT, b * s)
    out = gather(table, ids_flat)
    return out.reshape(b, s, _D_OUT)


# speedup vs baseline: 3.6544x; 1.2227x over previous
import functools

import jax
import jax.numpy as jnp
from jax import lax
from jax.experimental import pallas as pl
from jax.experimental.pallas import tpu as pltpu
from jax.experimental.pallas import tpu_sc as plsc

_EPS = 1e-06
_D_OUT = 2048


def _table_body(emb_ref, nw_ref, pw_ref, out_ref):
    emb = emb_ref[...]
    normed = emb * lax.rsqrt(jnp.mean(emb * emb, axis=-1, keepdims=True) + _EPS)
    normed = normed * nw_ref[...]
    proj = lax.dot_general(
        normed, pw_ref[...], (((1,), (1,)), ((), ())),
        preferred_element_type=jnp.float32)
    out_ref[...] = proj * lax.rsqrt(
        jnp.mean(proj * proj, axis=-1, keepdims=True) + _EPS)


def _make_gather(vocab, d, batch):
    info = plsc.get_sparse_core_info()
    nc, ns = info.num_cores, info.num_subcores
    nw = nc * ns
    assert batch % (8 * nw) == 0
    b_per_w = batch // nw
    chunk = 16
    assert b_per_w % (2 * chunk) == 0
    n_pairs = b_per_w // (2 * chunk)
    mesh = plsc.VectorSubcoreMesh(core_axis_name="c", subcore_axis_name="s")

    @functools.partial(
        pl.kernel,
        mesh=mesh,
        out_type=jax.ShapeDtypeStruct((batch, d), jnp.float32),
        scratch_types=[
            pltpu.VMEM((b_per_w,), jnp.int32),
            pltpu.VMEM((chunk, d), jnp.float32),
            pltpu.VMEM((chunk, d), jnp.float32),
            pltpu.SemaphoreType.DMA,
            pltpu.SemaphoreType.DMA,
            pltpu.SemaphoreType.DMA,
            pltpu.SemaphoreType.DMA,
        ],
    )
    def gather(table_hbm, ids_hbm, out_hbm, idx_v, buf0, buf1,
               gsem0, gsem1, wsem0, wsem1):
        wid = lax.axis_index("s") * nc + lax.axis_index("c")
        base = wid * b_per_w
        pltpu.sync_copy(ids_hbm.at[pl.ds(base, b_per_w)], idx_v)

        def start_gather(i, buf, gsem):
            pltpu.async_copy(
                table_hbm.at[idx_v.at[pl.ds(i * chunk, chunk)]], buf, gsem)

        def wait_gather(buf, gsem):
            pltpu.make_async_copy(
                table_hbm.at[idx_v.at[pl.ds(0, chunk)]], buf, gsem).wait()

        start_gather(0, buf0, gsem0)
        start_gather(1, buf1, gsem1)

        def body(p, _):
            for par, (buf, gsem, wsem) in enumerate(
                    ((buf0, gsem0, wsem0), (buf1, gsem1, wsem1))):
                i = p * 2 + par
                wait_gather(buf, gsem)
                pltpu.async_copy(
                    buf, out_hbm.at[pl.ds(base + i * chunk, chunk)], wsem
                ).wait()

                @pl.when(i + 2 < 2 * n_pairs)
                def _():
                    start_gather(i + 2, buf, gsem)
            return 0

        lax.fori_loop(0, n_pairs, body, 0)

    return gather


def kernel(input_ids, embedding, hard_norm_weight, proj_weight):
    vocab, mm_hidden = embedding.shape
    b, s = input_ids.shape
    table = pl.pallas_call(
        _table_body,
        out_shape=jax.ShapeDtypeStruct((vocab, _D_OUT), jnp.float32),
    )(embedding, hard_norm_weight.reshape(1, mm_hidden), proj_weight)
    n_workers = 32
    table_rep = jnp.broadcast_to(
        table, (n_workers, vocab, _D_OUT)).reshape(n_workers * vocab, _D_OUT)
    ids_flat = input_ids.reshape(b * s).astype(jnp.int32)
    per_w = (b * s) // n_workers
    ids_flat = ids_flat + jnp.repeat(
        jnp.arange(n_workers, dtype=jnp.int32) * vocab, per_w)
    gather = _make_gather(n_workers * vocab, _D_OUT, b * s)
    out = gather(table_rep, ids_flat)
    return out.reshape(b, s, _D_OUT)
